# baseline (speedup 1.0000x reference)
import functools

import jax
import jax.numpy as jnp
from jax import lax
from jax.experimental import pallas as pl
from jax.experimental.pallas import tpu as pltpu
from jax.experimental.pallas import tpu_sc as plsc

MEM_ROWS = 65536
BATCH = 4096
FEAT_DIM = 256
NUM_CORES = 2
NUM_SUBCORES = 16
NUM_WORKERS = NUM_CORES * NUM_SUBCORES
ROWS_PER_W = MEM_ROWS // NUM_WORKERS

CHUNK = 128
NCHUNK = ROWS_PER_W // CHUNK
NBUF = 3
AHEAD = 1


def kernel(x, feats):
    mesh = plsc.VectorSubcoreMesh(
        core_axis_name="core", subcore_axis_name="subcore"
    )

    @functools.partial(
        pl.kernel,
        out_type=jax.ShapeDtypeStruct((MEM_ROWS, FEAT_DIM), jnp.float32),
        mesh=mesh,
        scratch_types=[
            pltpu.VMEM((NBUF, CHUNK, FEAT_DIM), jnp.float32),
            pltpu.SemaphoreType.DMA((NBUF,)),
            pltpu.SemaphoreType.DMA((NBUF,)),
        ],
    )
    def bank(x_hbm, f_hbm, o_hbm, buf, gsem, ssem):
        wid = lax.axis_index("subcore") * NUM_CORES + lax.axis_index("core")
        base = wid * ROWS_PER_W

        def pump(src_hbm):
            gathers, scatters = [None] * NCHUNK, [None] * NCHUNK

            def fire_gather(i):
                b = i % NBUF
                if i >= NBUF:
                    scatters[i - NBUF].wait()
                gathers[i] = pltpu.make_async_copy(
                    src_hbm.at[pl.ds(base + i * CHUNK, CHUNK)],
                    buf.at[b], gsem.at[b])
                gathers[i].start()

            for i in range(AHEAD):
                fire_gather(i)
            for i in range(NCHUNK):
                if i + AHEAD < NCHUNK:
                    fire_gather(i + AHEAD)
                b = i % NBUF
                gathers[i].wait()
                scatters[i] = pltpu.make_async_copy(
                    buf.at[b],
                    o_hbm.at[pl.ds(base + i * CHUNK, CHUNK)], ssem.at[b])
                scatters[i].start()
            for i in range(NCHUNK - NBUF, NCHUNK):
                scatters[i].wait()

        @pl.when(base < BATCH)
        def _():
            pump(x_hbm)

        @pl.when(base >= BATCH)
        def _():
            pump(f_hbm)

    return bank(x, feats)


# speedup vs baseline: 1.0148x; 1.0148x over previous
import functools

import jax
import jax.numpy as jnp
from jax import lax
from jax.experimental import pallas as pl
from jax.experimental.pallas import tpu as pltpu
from jax.experimental.pallas import tpu_sc as plsc

MEM_ROWS = 65536
BATCH = 4096
FEAT_DIM = 256
NUM_CORES = 2
NUM_SUBCORES = 16
NUM_WORKERS = NUM_CORES * NUM_SUBCORES
ROWS_PER_W = MEM_ROWS // NUM_WORKERS

CHUNK = 64
NCHUNK = ROWS_PER_W // CHUNK
NBUF = 6
AHEAD = 3


def kernel(x, feats):
    mesh = plsc.VectorSubcoreMesh(
        core_axis_name="core", subcore_axis_name="subcore"
    )

    @functools.partial(
        pl.kernel,
        out_type=jax.ShapeDtypeStruct((MEM_ROWS, FEAT_DIM), jnp.float32),
        mesh=mesh,
        scratch_types=[
            pltpu.VMEM((NBUF, CHUNK, FEAT_DIM), jnp.float32),
            pltpu.SemaphoreType.DMA((NBUF,)),
            pltpu.SemaphoreType.DMA((NBUF,)),
        ],
    )
    def bank(x_hbm, f_hbm, o_hbm, buf, gsem, ssem):
        wid = lax.axis_index("subcore") * NUM_CORES + lax.axis_index("core")
        base = wid * ROWS_PER_W

        def pump(src_hbm):
            gathers, scatters = [None] * NCHUNK, [None] * NCHUNK

            def fire_gather(i):
                b = i % NBUF
                if i >= NBUF:
                    scatters[i - NBUF].wait()
                gathers[i] = pltpu.make_async_copy(
                    src_hbm.at[pl.ds(base + i * CHUNK, CHUNK)],
                    buf.at[b], gsem.at[b])
                gathers[i].start()

            for i in range(AHEAD):
                fire_gather(i)
            for i in range(NCHUNK):
                if i + AHEAD < NCHUNK:
                    fire_gather(i + AHEAD)
                b = i % NBUF
                gathers[i].wait()
                scatters[i] = pltpu.make_async_copy(
                    buf.at[b],
                    o_hbm.at[pl.ds(base + i * CHUNK, CHUNK)], ssem.at[b])
                scatters[i].start()
            for i in range(NCHUNK - NBUF, NCHUNK):
                scatters[i].wait()

        @pl.when(base < BATCH)
        def _():
            pump(x_hbm)

        @pl.when(base >= BATCH)
        def _():
            pump(f_hbm)

    return bank(x, feats)
